# baseline (speedup 1.0000x reference)
import dataclasses
import functools

import jax
import jax.numpy as jnp
from jax import lax
from jax.experimental import pallas as pl
from jax.experimental.pallas import tpu as pltpu
from jax.experimental.pallas import tpu_sc as plsc

W = 4
N = 10000
E = 160000
FI = 128
FH = 64
G4 = 4 * FH

NC = 2
NS = 16
NW = NC * NS
B = 128
NB = 40
EPW = NB * B
E2 = EPW * NW

ROW_STRIDE = 624
ROW_SPAN = 640
ZR = 128

_mesh = lambda: plsc.VectorSubcoreMesh(core_axis_name="c", subcore_axis_name="s")


def _sc_params():
    cp = pltpu.CompilerParams()
    cp = dataclasses.replace(cp, needs_layout_passes=False,
                             use_tc_tiling_on_sc=False)
    return cp


def _zero_fill(zbuf, rows, width):
    @pl.loop(0, rows)
    def _(r):
        for k in range(width // 16):
            zbuf[r, pl.ds(k * 16, 16)] = jnp.zeros((16,), jnp.float32)


def _lane(vec, rr):
    idx = jnp.full((16,), rr, jnp.int32)
    return vec.at[idx].get(mode="promise_in_bounds")


_RING = 8


@functools.partial(
    pl.kernel,
    mesh=_mesh(),
    compiler_params=_sc_params(),
    out_type=jax.ShapeDtypeStruct((W, NC, N, 16), jnp.float32),
    scratch_types=[
        pltpu.VMEM((NB, B), jnp.int32),
        pltpu.VMEM((EPW + 8,), jnp.float32),
        pltpu.VMEM((EPW, 16), jnp.float32),
        pltpu.VMEM((ZR, 16), jnp.float32),
        pltpu.VMEM_SHARED((N, 16), jnp.float32),
        pltpu.SemaphoreType.DMA,
    ],
)
def _deg_kernel(src_hbm, wflat_hbm, out_hbm, srcv, wflat, srows, zbuf, acc,
                ssem):
    cid = lax.axis_index("c")
    sid = lax.axis_index("s")
    wid = sid * NC + cid
    _zero_fill(zbuf, ZR, 16)
    base = sid * ROW_STRIDE

    @pl.loop(0, W)
    def _(t):
        for z in range(ROW_SPAN // ZR):
            pltpu.sync_copy(zbuf, acc.at[pl.ds(base + z * ZR, ZR)])
        pltpu.sync_copy(src_hbm.at[t, wid], srcv)
        pltpu.sync_copy(wflat_hbm.at[t, wid], wflat.at[pl.ds(0, EPW)])
        plsc.subcore_barrier()

        @pl.loop(0, EPW, step=8)
        def _(r0):
            wrow = wflat[pl.ds(r0, 16)]
            for rr in range(8):
                srows[r0 + rr, pl.ds(0, 16)] = _lane(wrow, rr)

        def scat(j):
            return (srows.at[pl.ds(j * B, B)], acc.at[srcv.at[j]])

        @pl.loop(0, NB)
        def _(j):
            s, d = scat(j)
            pltpu.async_copy(s, d, ssem, add=True)

            @pl.when(j >= _RING)
            def _():
                s2, d2 = scat(j - _RING)
                pltpu.make_async_copy(s2, d2, ssem).wait()

        @pl.loop(NB - _RING, NB)
        def _(j):
            s, d = scat(j)
            pltpu.make_async_copy(s, d, ssem).wait()

        plsc.subcore_barrier()
        pltpu.sync_copy(acc.at[pl.ds(base, ROW_SPAN)],
                        out_hbm.at[t, cid, pl.ds(base, ROW_SPAN)])
        plsc.subcore_barrier()


def _make_seg_kernel(nt, nts, edge_t_fn, feat, half_pair=False):
    out_shape = ((nt // 2, NC, N, 2, feat) if half_pair
                 else (nt, NC, N, feat))

    @functools.partial(
        pl.kernel,
        mesh=_mesh(),
        compiler_params=_sc_params(),
        out_type=jax.ShapeDtypeStruct(out_shape, jnp.float32),
        scratch_types=[
            pltpu.VMEM((NB, B), jnp.int32),
            pltpu.VMEM((NB, B), jnp.int32),
            pltpu.VMEM((EPW + 8,), jnp.float32),
            pltpu.VMEM((B, feat), jnp.float32),
            pltpu.VMEM((B, feat), jnp.float32),
            pltpu.VMEM((B, feat), jnp.float32),
            pltpu.VMEM((B, feat), jnp.float32),
            pltpu.VMEM((ZR, feat), jnp.float32),
            pltpu.VMEM_SHARED((N, feat), jnp.float32),
            pltpu.SemaphoreType.DMA,
            pltpu.SemaphoreType.DMA,
            pltpu.SemaphoreType.DMA,
            pltpu.SemaphoreType.DMA,
        ],
    )
    def _seg_kernel(table_hbm, src_hbm, dst_hbm, w_hbm, out_hbm,
                    srcv, dstv, wfl, g0, g1, s0, s1, zbuf, acc,
                    gsem0, gsem1, ssem0, ssem1):
        cid = lax.axis_index("c")
        sid = lax.axis_index("s")
        wid = sid * NC + cid
        _zero_fill(zbuf, ZR, feat)
        base = sid * ROW_STRIDE

        @pl.loop(0, nt)
        def _(ti):
            t = edge_t_fn(ti)
            for z in range(ROW_SPAN // ZR):
                pltpu.sync_copy(zbuf, acc.at[pl.ds(base + z * ZR, ZR)])
            pltpu.sync_copy(src_hbm.at[t, wid], srcv)
            pltpu.sync_copy(dst_hbm.at[t, wid], dstv)
            pltpu.sync_copy(w_hbm.at[t, wid], wfl.at[pl.ds(0, EPW)])
            plsc.subcore_barrier()

            def g_pair(jj, gb):
                return (table_hbm.at[ti].at[srcv.at[jj]], gb)

            def s_pair(jj, sb):
                return (sb, acc.at[dstv.at[jj]])

            def issue_g(jj, gb, gsem):
                s, d = g_pair(jj, gb)
                pltpu.async_copy(s, d, gsem)

            def wait_g(jj, gb, gsem):
                s, d = g_pair(jj, gb)
                pltpu.make_async_copy(s, d, gsem).wait()

            def issue_s(jj, sb, ssem):
                s, d = s_pair(jj, sb)
                pltpu.async_copy(s, d, ssem, add=True)

            def wait_s(jj, sb, ssem):
                s, d = s_pair(jj, sb)
                pltpu.make_async_copy(s, d, ssem).wait()

            def scale(jj, gb, sb):
                @pl.loop(0, B, step=16)
                def _(r0):
                    wrow = wfl[pl.ds(jj * B + r0, 16)]
                    for rr in range(16):
                        wvec = _lane(wrow, rr)
                        for k in range(feat // 16):
                            sl = pl.ds(k * 16, 16)
                            sb[r0 + rr, sl] = gb[r0 + rr, sl] * wvec

            issue_g(0, g0, gsem0)
            issue_g(1, g1, gsem1)
            wait_g(0, g0, gsem0)
            scale(0, g0, s0)
            issue_s(0, s0, ssem0)
            issue_g(2, g0, gsem0)
            wait_g(1, g1, gsem1)
            scale(1, g1, s1)
            issue_s(1, s1, ssem1)
            issue_g(3, g1, gsem1)

            @pl.loop(2, NB, step=2)
            def _(j):
                wait_g(j, g0, gsem0)
                wait_s(j - 2, s0, ssem0)
                scale(j, g0, s0)
                issue_s(j, s0, ssem0)

                @pl.when(j + 2 < NB)
                def _():
                    issue_g(j + 2, g0, gsem0)

                wait_g(j + 1, g1, gsem1)
                wait_s(j - 1, s1, ssem1)
                scale(j + 1, g1, s1)
                issue_s(j + 1, s1, ssem1)

                @pl.when(j + 3 < NB)
                def _():
                    issue_g(j + 3, g1, gsem1)

            wait_s(NB - 2, s0, ssem0)
            wait_s(NB - 1, s1, ssem1)

            plsc.subcore_barrier()
            if half_pair:
                pltpu.sync_copy(
                    acc.at[pl.ds(base, ROW_SPAN)],
                    out_hbm.at[ti // 2, cid, pl.ds(base, ROW_SPAN), ti % 2])
            else:
                pltpu.sync_copy(acc.at[pl.ds(base, ROW_SPAN)],
                                out_hbm.at[ti, cid, pl.ds(base, ROW_SPAN)])
            plsc.subcore_barrier()

    return _seg_kernel


_xside_kernel = _make_seg_kernel(2 * W, W, lambda ti: ti // 2, FH,
                                 half_pair=True)
_hside_kernel = _make_seg_kernel(1, 1, lambda ti: ti, FH)


BN = 1000


def _prep_body(dp_ref, x_ref, dis_ref, xs_ref):
    dp = dp_ref[...]
    deg = dp[0, 0] + dp[0, 1]
    dis = jnp.where(deg > 0.0,
                    lax.rsqrt(jnp.maximum(deg, 1e-12)),
                    jnp.zeros_like(deg))
    dcol = dis[:, 0:1]
    dis_ref[...] = dcol.reshape(1, BN, 1)
    xb = x_ref[...][0]
    xs_ref[0] = dcol * xb[:, 0:FH]
    xs_ref[1] = dcol * xb[:, FH:FI]


def _prep_call(degparts, x):
    return pl.pallas_call(
        _prep_body,
        grid=(W, N // BN),
        in_specs=[
            pl.BlockSpec((1, NC, BN, 16), lambda t, i: (t, 0, i, 0)),
            pl.BlockSpec((1, BN, FI), lambda t, i: (t, i, 0)),
        ],
        out_specs=[
            pl.BlockSpec((1, BN, 1), lambda t, i: (t, i, 0)),
            pl.BlockSpec((2, BN, FH), lambda t, i: (t, i, 0)),
        ],
        out_shape=[
            jax.ShapeDtypeStruct((W, N, 1), jnp.float32),
            jax.ShapeDtypeStruct((2 * W, N, FH), jnp.float32),
        ],
    )(degparts, x)


def _gate_body(x_ref, u_ref, v_ref, h_ref, c_ref, dis_ref, disn_ref,
               wx0_ref, wx1_ref, wh0_ref, wh1_ref, b_ref, wc_ref,
               hn_ref, cn_ref, hs_ref):
    d = dis_ref[...]
    u = -d * (u_ref[0] + u_ref[1])
    v = -d * (v_ref[0, 0] + v_ref[0, 1])
    f32 = jnp.float32
    z = (jnp.dot(x_ref[...], wx0_ref[...], preferred_element_type=f32)
         + jnp.dot(u, wx1_ref[...], preferred_element_type=f32)
         + jnp.dot(h_ref[...], wh0_ref[...], preferred_element_type=f32)
         + jnp.dot(v, wh1_ref[...], preferred_element_type=f32)
         + b_ref[...])
    c_old = c_ref[...]
    ig = jax.nn.sigmoid(z[:, 0:FH] + wc_ref[0:1, :] * c_old)
    fg = jax.nn.sigmoid(z[:, FH:2 * FH] + wc_ref[1:2, :] * c_old)
    tg = jnp.tanh(z[:, 2 * FH:3 * FH])
    cn = fg * c_old + ig * tg
    og = jax.nn.sigmoid(z[:, 3 * FH:4 * FH] + wc_ref[2:3, :] * cn)
    hn = og * jnp.tanh(cn)
    hn_ref[...] = hn
    cn_ref[...] = cn
    hs_ref[...] = disn_ref[...] * hn


def _gate_call(x_t, uparts, vparts, h, c, dis_t, dis_n, wx0, wx1,
               wh0, wh1, bias, wc3):
    whole = lambda shp: pl.BlockSpec(shp, lambda i: tuple(0 for _ in shp))
    row = lambda f: pl.BlockSpec((BN, f), lambda i: (i, 0))
    return pl.pallas_call(
        _gate_body,
        grid=(N // BN,),
        in_specs=[
            row(FI),
            pl.BlockSpec((NC, BN, FI), lambda i: (0, i, 0)),
            pl.BlockSpec((1, NC, BN, FH), lambda i: (0, 0, i, 0)),
            row(FH), row(FH), row(1), row(1),
            whole((FI, G4)), whole((FI, G4)),
            whole((FH, G4)), whole((FH, G4)),
            whole((1, G4)), whole((3, FH)),
        ],
        out_specs=[row(FH), row(FH), row(FH)],
        out_shape=[jax.ShapeDtypeStruct((N, FH), jnp.float32)] * 3,
    )(x_t, uparts, vparts, h, c, dis_t, dis_n, wx0, wx1, wh0, wh1,
      bias, wc3)


def _final_body(h_ref, lw_ref, lb_ref, o_ref):
    o_ref[...] = (jnp.dot(h_ref[...], lw_ref[...],
                          preferred_element_type=jnp.float32)
                  + lb_ref[...])


def _final_call(h, lin_W, lin_b):
    return pl.pallas_call(
        _final_body,
        grid=(N // BN,),
        in_specs=[
            pl.BlockSpec((BN, FH), lambda i: (i, 0)),
            pl.BlockSpec((FH, 1), lambda i: (0, 0)),
            pl.BlockSpec((1, 1), lambda i: (0, 0)),
        ],
        out_specs=pl.BlockSpec((BN, 1), lambda i: (i, 0)),
        out_shape=jax.ShapeDtypeStruct((N, 1), jnp.float32),
    )(h, lin_W, lin_b.reshape(1, 1))


def kernel(x, edge_index, edge_weight, Wx, Wh, bx, bh, wc, b, lin_W, lin_b):
    pad = E2 - E
    pidx = jnp.broadcast_to(jnp.arange(pad, dtype=jnp.int32) % N, (W, pad))
    src_r = jnp.concatenate([edge_index[:, 0, :], pidx],
                            axis=1).reshape(W, NW, NB, B)
    dst_r = jnp.concatenate([edge_index[:, 1, :], pidx],
                            axis=1).reshape(W, NW, NB, B)
    w_r = jnp.concatenate(
        [edge_weight, jnp.zeros((W, pad), jnp.float32)],
        axis=1).reshape(W, NW, EPW)

    wx0 = jnp.transpose(Wx[:, 0], (1, 0, 2)).reshape(FI, G4)
    wx1 = jnp.transpose(Wx[:, 1], (1, 0, 2)).reshape(FI, G4)
    wh0 = jnp.transpose(Wh[:, 0], (1, 0, 2)).reshape(FH, G4)
    wh1 = jnp.transpose(Wh[:, 1], (1, 0, 2)).reshape(FH, G4)
    bias = (bx + bh + b).reshape(1, G4)

    degparts = _deg_kernel(src_r, w_r)
    dis, xs = _prep_call(degparts, x)
    uparts = _xside_kernel(xs, src_r, dst_r, w_r)
    uparts = uparts.reshape(W, NC, N, FI)

    h = jnp.zeros((N, FH), jnp.float32)
    c = jnp.zeros((N, FH), jnp.float32)
    vzero = jnp.zeros((1, NC, N, FH), jnp.float32)
    hs = None
    for t in range(W):
        if t == 0:
            vparts = vzero
        else:
            vparts = _hside_kernel(hs[None], src_r[t:t + 1], dst_r[t:t + 1],
                                   w_r[t:t + 1])
        dis_n = dis[min(t + 1, W - 1)]
        h, c, hs = _gate_call(x[t], uparts[t], vparts, h, c,
                              dis[t], dis_n, wx0, wx1, wh0, wh1,
                              bias, wc)
    out = _final_call(h, lin_W, lin_b)
    return (out.reshape(N), h, c)


# speedup vs baseline: 1.1558x; 1.1558x over previous
import dataclasses
import functools

import jax
import jax.numpy as jnp
from jax import lax
from jax.experimental import pallas as pl
from jax.experimental.pallas import tpu as pltpu
from jax.experimental.pallas import tpu_sc as plsc

W = 4
N = 10000
E = 160000
FI = 128
FH = 64
G4 = 4 * FH

NC = 2
NS = 16
NW = NC * NS
B = 128
NB = 40
EPW = NB * B
E2 = EPW * NW

ROW_STRIDE = 624
ROW_SPAN = 640
ZR = 128

_mesh = lambda: plsc.VectorSubcoreMesh(core_axis_name="c", subcore_axis_name="s")


def _sc_params():
    cp = pltpu.CompilerParams()
    cp = dataclasses.replace(cp, needs_layout_passes=False,
                             use_tc_tiling_on_sc=False)
    return cp


def _zero_fill(zbuf, rows, width):
    @pl.loop(0, rows)
    def _(r):
        for k in range(width // 16):
            zbuf[r, pl.ds(k * 16, 16)] = jnp.zeros((16,), jnp.float32)


def _lane(vec, rr):
    idx = jnp.full((16,), rr, jnp.int32)
    return vec.at[idx].get(mode="promise_in_bounds")


_RING = 8


@functools.partial(
    pl.kernel,
    mesh=_mesh(),
    compiler_params=_sc_params(),
    out_type=jax.ShapeDtypeStruct((W, NC, N, 16), jnp.float32),
    scratch_types=[
        pltpu.VMEM((NB, B), jnp.int32),
        pltpu.VMEM((EPW + 8,), jnp.float32),
        pltpu.VMEM((EPW, 16), jnp.float32),
        pltpu.VMEM((ZR, 16), jnp.float32),
        pltpu.VMEM_SHARED((N, 16), jnp.float32),
        pltpu.SemaphoreType.DMA,
    ],
)
def _deg_kernel(src_hbm, wflat_hbm, out_hbm, srcv, wflat, srows, zbuf, acc,
                ssem):
    cid = lax.axis_index("c")
    sid = lax.axis_index("s")
    wid = sid * NC + cid
    _zero_fill(zbuf, ZR, 16)
    base = sid * ROW_STRIDE

    @pl.loop(0, W)
    def _(t):
        for z in range(ROW_SPAN // ZR):
            pltpu.sync_copy(zbuf, acc.at[pl.ds(base + z * ZR, ZR)])
        pltpu.sync_copy(src_hbm.at[t, wid], srcv)
        pltpu.sync_copy(wflat_hbm.at[t, wid], wflat.at[pl.ds(0, EPW)])
        plsc.subcore_barrier()

        @pl.loop(0, EPW, step=8)
        def _(r0):
            wrow = wflat[pl.ds(r0, 16)]
            for rr in range(8):
                srows[r0 + rr, pl.ds(0, 16)] = _lane(wrow, rr)

        def scat(j):
            return (srows.at[pl.ds(j * B, B)], acc.at[srcv.at[j]])

        @pl.loop(0, NB)
        def _(j):
            s, d = scat(j)
            pltpu.async_copy(s, d, ssem, add=True)

            @pl.when(j >= _RING)
            def _():
                s2, d2 = scat(j - _RING)
                pltpu.make_async_copy(s2, d2, ssem).wait()

        @pl.loop(NB - _RING, NB)
        def _(j):
            s, d = scat(j)
            pltpu.make_async_copy(s, d, ssem).wait()

        plsc.subcore_barrier()
        pltpu.sync_copy(acc.at[pl.ds(base, ROW_SPAN)],
                        out_hbm.at[t, cid, pl.ds(base, ROW_SPAN)])
        plsc.subcore_barrier()


def _make_seg_kernel(nt, nts, edge_t_fn, feat):
    @functools.partial(
        pl.kernel,
        mesh=_mesh(),
        compiler_params=_sc_params(),
        out_type=jax.ShapeDtypeStruct((nt, NC, N, feat), jnp.float32),
        scratch_types=[
            pltpu.VMEM((NB, B), jnp.int32),
            pltpu.VMEM((NB, B), jnp.int32),
            pltpu.VMEM((EPW + 8,), jnp.float32),
            pltpu.VMEM((B, feat), jnp.float32),
            pltpu.VMEM((B, feat), jnp.float32),
            pltpu.VMEM((B, feat), jnp.float32),
            pltpu.VMEM((B, feat), jnp.float32),
            pltpu.VMEM((ZR, feat), jnp.float32),
            pltpu.VMEM_SHARED((N, feat), jnp.float32),
            pltpu.SemaphoreType.DMA,
            pltpu.SemaphoreType.DMA,
            pltpu.SemaphoreType.DMA,
            pltpu.SemaphoreType.DMA,
        ],
    )
    def _seg_kernel(table_hbm, src_hbm, dst_hbm, w_hbm, out_hbm,
                    srcv, dstv, wfl, g0, g1, s0, s1, zbuf, acc,
                    gsem0, gsem1, ssem0, ssem1):
        cid = lax.axis_index("c")
        sid = lax.axis_index("s")
        wid = sid * NC + cid
        _zero_fill(zbuf, ZR, feat)
        base = sid * ROW_STRIDE

        @pl.loop(0, nt)
        def _(ti):
            t = edge_t_fn(ti)
            for z in range(ROW_SPAN // ZR):
                pltpu.sync_copy(zbuf, acc.at[pl.ds(base + z * ZR, ZR)])
            pltpu.sync_copy(src_hbm.at[t, wid], srcv)
            pltpu.sync_copy(dst_hbm.at[t, wid], dstv)
            pltpu.sync_copy(w_hbm.at[t, wid], wfl.at[pl.ds(0, EPW)])
            plsc.subcore_barrier()

            def g_pair(jj, gb):
                return (table_hbm.at[ti].at[srcv.at[jj]], gb)

            def s_pair(jj, sb):
                return (sb, acc.at[dstv.at[jj]])

            def issue_g(jj, gb, gsem):
                s, d = g_pair(jj, gb)
                pltpu.async_copy(s, d, gsem)

            def wait_g(jj, gb, gsem):
                s, d = g_pair(jj, gb)
                pltpu.make_async_copy(s, d, gsem).wait()

            def issue_s(jj, sb, ssem):
                s, d = s_pair(jj, sb)
                pltpu.async_copy(s, d, ssem, add=True)

            def wait_s(jj, sb, ssem):
                s, d = s_pair(jj, sb)
                pltpu.make_async_copy(s, d, ssem).wait()

            def scale(jj, gb, sb):
                @pl.loop(0, B, step=16)
                def _(r0):
                    wrow = wfl[pl.ds(jj * B + r0, 16)]
                    for rr in range(16):
                        wvec = _lane(wrow, rr)
                        for k in range(feat // 16):
                            sl = pl.ds(k * 16, 16)
                            sb[r0 + rr, sl] = gb[r0 + rr, sl] * wvec

            issue_g(0, g0, gsem0)
            issue_g(1, g1, gsem1)
            wait_g(0, g0, gsem0)
            scale(0, g0, s0)
            issue_s(0, s0, ssem0)
            issue_g(2, g0, gsem0)
            wait_g(1, g1, gsem1)
            scale(1, g1, s1)
            issue_s(1, s1, ssem1)
            issue_g(3, g1, gsem1)

            @pl.loop(2, NB, step=2)
            def _(j):
                wait_g(j, g0, gsem0)
                wait_s(j - 2, s0, ssem0)
                scale(j, g0, s0)
                issue_s(j, s0, ssem0)

                @pl.when(j + 2 < NB)
                def _():
                    issue_g(j + 2, g0, gsem0)

                wait_g(j + 1, g1, gsem1)
                wait_s(j - 1, s1, ssem1)
                scale(j + 1, g1, s1)
                issue_s(j + 1, s1, ssem1)

                @pl.when(j + 3 < NB)
                def _():
                    issue_g(j + 3, g1, gsem1)

            wait_s(NB - 2, s0, ssem0)
            wait_s(NB - 1, s1, ssem1)

            plsc.subcore_barrier()
            pltpu.sync_copy(acc.at[pl.ds(base, ROW_SPAN)],
                            out_hbm.at[ti, cid, pl.ds(base, ROW_SPAN)])
            plsc.subcore_barrier()

    return _seg_kernel


_xside_kernel = _make_seg_kernel(2 * W, W, lambda ti: ti // 2, FH)
_hside_kernel = _make_seg_kernel(1, 1, lambda ti: ti, FH)


BN = 1000


def _prep_body(dp_ref, x_ref, dis_ref, xs_ref):
    dp = dp_ref[...]
    deg = dp[0, 0] + dp[0, 1]
    dis = jnp.where(deg > 0.0,
                    lax.rsqrt(jnp.maximum(deg, 1e-12)),
                    jnp.zeros_like(deg))
    dcol = dis[:, 0:1]
    dis_ref[...] = dcol.reshape(1, BN, 1)
    xb = x_ref[...][0]
    xs_ref[0] = dcol * xb[:, 0:FH]
    xs_ref[1] = dcol * xb[:, FH:FI]


def _prep_call(degparts, x):
    return pl.pallas_call(
        _prep_body,
        grid=(W, N // BN),
        in_specs=[
            pl.BlockSpec((1, NC, BN, 16), lambda t, i: (t, 0, i, 0)),
            pl.BlockSpec((1, BN, FI), lambda t, i: (t, i, 0)),
        ],
        out_specs=[
            pl.BlockSpec((1, BN, 1), lambda t, i: (t, i, 0)),
            pl.BlockSpec((2, BN, FH), lambda t, i: (t, i, 0)),
        ],
        out_shape=[
            jax.ShapeDtypeStruct((W, N, 1), jnp.float32),
            jax.ShapeDtypeStruct((2 * W, N, FH), jnp.float32),
        ],
    )(degparts, x)


def _gate_body(x_ref, u_ref, v_ref, h_ref, c_ref, dis_ref, disn_ref,
               wx0_ref, wx1_ref, wh0_ref, wh1_ref, b_ref, wc_ref,
               hn_ref, cn_ref, hs_ref):
    d = dis_ref[...]
    ua = -d * (u_ref[0, 0] + u_ref[0, 1])
    ub = -d * (u_ref[1, 0] + u_ref[1, 1])
    v = -d * (v_ref[0, 0] + v_ref[0, 1])
    f32 = jnp.float32
    z = (jnp.dot(x_ref[...], wx0_ref[...], preferred_element_type=f32)
         + jnp.dot(ua, wx1_ref[...][:FH], preferred_element_type=f32)
         + jnp.dot(ub, wx1_ref[...][FH:], preferred_element_type=f32)
         + jnp.dot(h_ref[...], wh0_ref[...], preferred_element_type=f32)
         + jnp.dot(v, wh1_ref[...], preferred_element_type=f32)
         + b_ref[...])
    c_old = c_ref[...]
    ig = jax.nn.sigmoid(z[:, 0:FH] + wc_ref[0:1, :] * c_old)
    fg = jax.nn.sigmoid(z[:, FH:2 * FH] + wc_ref[1:2, :] * c_old)
    tg = jnp.tanh(z[:, 2 * FH:3 * FH])
    cn = fg * c_old + ig * tg
    og = jax.nn.sigmoid(z[:, 3 * FH:4 * FH] + wc_ref[2:3, :] * cn)
    hn = og * jnp.tanh(cn)
    hn_ref[...] = hn
    cn_ref[...] = cn
    hs_ref[...] = disn_ref[...] * hn


def _gate_call(x_t, uparts, vparts, h, c, dis_t, dis_n, wx0, wx1,
               wh0, wh1, bias, wc3):
    whole = lambda shp: pl.BlockSpec(shp, lambda i: tuple(0 for _ in shp))
    row = lambda f: pl.BlockSpec((BN, f), lambda i: (i, 0))
    return pl.pallas_call(
        _gate_body,
        grid=(N // BN,),
        in_specs=[
            row(FI),
            pl.BlockSpec((2, NC, BN, FH), lambda i: (0, 0, i, 0)),
            pl.BlockSpec((1, NC, BN, FH), lambda i: (0, 0, i, 0)),
            row(FH), row(FH), row(1), row(1),
            whole((FI, G4)), whole((FI, G4)),
            whole((FH, G4)), whole((FH, G4)),
            whole((1, G4)), whole((3, FH)),
        ],
        out_specs=[row(FH), row(FH), row(FH)],
        out_shape=[jax.ShapeDtypeStruct((N, FH), jnp.float32)] * 3,
    )(x_t, uparts, vparts, h, c, dis_t, dis_n, wx0, wx1, wh0, wh1,
      bias, wc3)


def _final_body(h_ref, lw_ref, lb_ref, o_ref):
    o_ref[...] = (jnp.dot(h_ref[...], lw_ref[...],
                          preferred_element_type=jnp.float32)
                  + lb_ref[...])


def _final_call(h, lin_W, lin_b):
    return pl.pallas_call(
        _final_body,
        grid=(N // BN,),
        in_specs=[
            pl.BlockSpec((BN, FH), lambda i: (i, 0)),
            pl.BlockSpec((FH, 1), lambda i: (0, 0)),
            pl.BlockSpec((1, 1), lambda i: (0, 0)),
        ],
        out_specs=pl.BlockSpec((BN, 1), lambda i: (i, 0)),
        out_shape=jax.ShapeDtypeStruct((N, 1), jnp.float32),
    )(h, lin_W, lin_b.reshape(1, 1))


def kernel(x, edge_index, edge_weight, Wx, Wh, bx, bh, wc, b, lin_W, lin_b):
    pad = E2 - E
    pidx = jnp.broadcast_to(jnp.arange(pad, dtype=jnp.int32) % N, (W, pad))
    src_r = jnp.concatenate([edge_index[:, 0, :], pidx],
                            axis=1).reshape(W, NW, NB, B)
    dst_r = jnp.concatenate([edge_index[:, 1, :], pidx],
                            axis=1).reshape(W, NW, NB, B)
    w_r = jnp.concatenate(
        [edge_weight, jnp.zeros((W, pad), jnp.float32)],
        axis=1).reshape(W, NW, EPW)

    wx0 = jnp.transpose(Wx[:, 0], (1, 0, 2)).reshape(FI, G4)
    wx1 = jnp.transpose(Wx[:, 1], (1, 0, 2)).reshape(FI, G4)
    wh0 = jnp.transpose(Wh[:, 0], (1, 0, 2)).reshape(FH, G4)
    wh1 = jnp.transpose(Wh[:, 1], (1, 0, 2)).reshape(FH, G4)
    bias = (bx + bh + b).reshape(1, G4)

    degparts = _deg_kernel(src_r, w_r)
    dis, xs = _prep_call(degparts, x)
    uparts = _xside_kernel(xs, src_r, dst_r, w_r)

    h = jnp.zeros((N, FH), jnp.float32)
    c = jnp.zeros((N, FH), jnp.float32)
    vzero = jnp.zeros((1, NC, N, FH), jnp.float32)
    hs = None
    for t in range(W):
        if t == 0:
            vparts = vzero
        else:
            vparts = _hside_kernel(hs[None], src_r[t:t + 1], dst_r[t:t + 1],
                                   w_r[t:t + 1])
        dis_n = dis[min(t + 1, W - 1)]
        h, c, hs = _gate_call(x[t], uparts[2 * t:2 * t + 2], vparts, h, c,
                              dis[t], dis_n, wx0, wx1, wh0, wh1,
                              bias, wc)
    out = _final_call(h, lin_W, lin_b)
    return (out.reshape(N), h, c)


# speedup vs baseline: 1.1957x; 1.0345x over previous
import dataclasses
import functools

import jax
import jax.numpy as jnp
from jax import lax
from jax.experimental import pallas as pl
from jax.experimental.pallas import tpu as pltpu
from jax.experimental.pallas import tpu_sc as plsc

W = 4
N = 10000
E = 160000
FI = 128
FH = 64
G4 = 4 * FH

NC = 2
NS = 16
NW = NC * NS
B = 128
NB = 40
EPW = NB * B
E2 = EPW * NW

ROW_STRIDE = 624
ROW_SPAN = 640
ZR = 128

_mesh = lambda: plsc.VectorSubcoreMesh(core_axis_name="c", subcore_axis_name="s")


def _sc_params():
    cp = pltpu.CompilerParams()
    cp = dataclasses.replace(cp, needs_layout_passes=False,
                             use_tc_tiling_on_sc=False)
    return cp


def _zero_fill(zbuf, rows, width):
    @pl.loop(0, rows)
    def _(r):
        for k in range(width // 16):
            zbuf[r, pl.ds(k * 16, 16)] = jnp.zeros((16,), jnp.float32)


def _lane(vec, rr):
    idx = jnp.full((16,), rr, jnp.int32)
    return vec.at[idx].get(mode="promise_in_bounds")


_RING = 8


@functools.partial(
    pl.kernel,
    mesh=_mesh(),
    compiler_params=_sc_params(),
    out_type=jax.ShapeDtypeStruct((W, NC, N, 16), jnp.float32),
    scratch_types=[
        pltpu.VMEM((NB, B), jnp.int32),
        pltpu.VMEM((EPW + 8,), jnp.float32),
        pltpu.VMEM((EPW, 16), jnp.float32),
        pltpu.VMEM((ZR, 16), jnp.float32),
        pltpu.VMEM_SHARED((N, 16), jnp.float32),
        pltpu.SemaphoreType.DMA,
    ],
)
def _deg_kernel(src_hbm, wflat_hbm, out_hbm, srcv, wflat, srows, zbuf, acc,
                ssem):
    cid = lax.axis_index("c")
    sid = lax.axis_index("s")
    wid = sid * NC + cid
    _zero_fill(zbuf, ZR, 16)
    base = sid * ROW_STRIDE

    @pl.loop(0, W)
    def _(t):
        for z in range(ROW_SPAN // ZR):
            pltpu.sync_copy(zbuf, acc.at[pl.ds(base + z * ZR, ZR)])
        pltpu.sync_copy(src_hbm.at[t, wid], srcv)
        pltpu.sync_copy(wflat_hbm.at[t, wid], wflat.at[pl.ds(0, EPW)])
        plsc.subcore_barrier()

        @pl.loop(0, EPW, step=8)
        def _(r0):
            wrow = wflat[pl.ds(r0, 16)]
            for rr in range(8):
                srows[r0 + rr, pl.ds(0, 16)] = _lane(wrow, rr)

        def scat(j):
            return (srows.at[pl.ds(j * B, B)], acc.at[srcv.at[j]])

        @pl.loop(0, NB)
        def _(j):
            s, d = scat(j)
            pltpu.async_copy(s, d, ssem, add=True)

            @pl.when(j >= _RING)
            def _():
                s2, d2 = scat(j - _RING)
                pltpu.make_async_copy(s2, d2, ssem).wait()

        @pl.loop(NB - _RING, NB)
        def _(j):
            s, d = scat(j)
            pltpu.make_async_copy(s, d, ssem).wait()

        plsc.subcore_barrier()
        pltpu.sync_copy(acc.at[pl.ds(base, ROW_SPAN)],
                        out_hbm.at[t, cid, pl.ds(base, ROW_SPAN)])
        plsc.subcore_barrier()


def _make_seg_kernel(nt, nts, edge_t_fn, feat):
    @functools.partial(
        pl.kernel,
        mesh=_mesh(),
        compiler_params=_sc_params(),
        out_type=jax.ShapeDtypeStruct((nt, NC, N, feat), jnp.float32),
        scratch_types=[
            pltpu.VMEM((NB, B), jnp.int32),
            pltpu.VMEM((NB, B), jnp.int32),
            pltpu.VMEM((EPW + 8,), jnp.float32),
            pltpu.VMEM((B, feat), jnp.float32),
            pltpu.VMEM((B, feat), jnp.float32),
            pltpu.VMEM((B, feat), jnp.float32),
            pltpu.VMEM((B, feat), jnp.float32),
            pltpu.VMEM((B, feat), jnp.float32),
            pltpu.VMEM((B, feat), jnp.float32),
            pltpu.VMEM((B, feat), jnp.float32),
            pltpu.VMEM((B, feat), jnp.float32),
            pltpu.VMEM((ZR, feat), jnp.float32),
            pltpu.VMEM_SHARED((N, feat), jnp.float32),
            pltpu.SemaphoreType.DMA,
            pltpu.SemaphoreType.DMA,
            pltpu.SemaphoreType.DMA,
            pltpu.SemaphoreType.DMA,
            pltpu.SemaphoreType.DMA,
            pltpu.SemaphoreType.DMA,
            pltpu.SemaphoreType.DMA,
            pltpu.SemaphoreType.DMA,
        ],
    )
    def _seg_kernel(table_hbm, src_hbm, dst_hbm, w_hbm, out_hbm,
                    srcv, dstv, wfl, g0, g1, g2, g3, s0, s1, s2, s3,
                    zbuf, acc, gsem0, gsem1, gsem2, gsem3,
                    ssem0, ssem1, ssem2, ssem3):
        cid = lax.axis_index("c")
        sid = lax.axis_index("s")
        wid = sid * NC + cid
        _zero_fill(zbuf, ZR, feat)
        base = sid * ROW_STRIDE

        @pl.loop(0, nt)
        def _(ti):
            t = edge_t_fn(ti)
            for z in range(ROW_SPAN // ZR):
                pltpu.sync_copy(zbuf, acc.at[pl.ds(base + z * ZR, ZR)])
            pltpu.sync_copy(src_hbm.at[t, wid], srcv)
            pltpu.sync_copy(dst_hbm.at[t, wid], dstv)
            pltpu.sync_copy(w_hbm.at[t, wid], wfl.at[pl.ds(0, EPW)])
            plsc.subcore_barrier()

            def g_pair(jj, gb):
                return (table_hbm.at[ti].at[srcv.at[jj]], gb)

            def s_pair(jj, sb):
                return (sb, acc.at[dstv.at[jj]])

            def issue_g(jj, gb, gsem):
                s, d = g_pair(jj, gb)
                pltpu.async_copy(s, d, gsem)

            def wait_g(jj, gb, gsem):
                s, d = g_pair(jj, gb)
                pltpu.make_async_copy(s, d, gsem).wait()

            def issue_s(jj, sb, ssem):
                s, d = s_pair(jj, sb)
                pltpu.async_copy(s, d, ssem, add=True)

            def wait_s(jj, sb, ssem):
                s, d = s_pair(jj, sb)
                pltpu.make_async_copy(s, d, ssem).wait()

            def scale(jj, gb, sb):
                @pl.loop(0, B, step=16)
                def _(r0):
                    wrow = wfl[pl.ds(jj * B + r0, 16)]
                    for rr in range(16):
                        wvec = _lane(wrow, rr)
                        for k in range(feat // 16):
                            sl = pl.ds(k * 16, 16)
                            sb[r0 + rr, sl] = gb[r0 + rr, sl] * wvec

            gbufs = (g0, g1, g2, g3)
            sbufs = (s0, s1, s2, s3)
            gsems = (gsem0, gsem1, gsem2, gsem3)
            ssems = (ssem0, ssem1, ssem2, ssem3)
            ND = 4

            for b in range(ND):
                issue_g(b, gbufs[b], gsems[b])
            for b in range(ND):
                wait_g(b, gbufs[b], gsems[b])
                scale(b, gbufs[b], sbufs[b])
                issue_s(b, sbufs[b], ssems[b])
                issue_g(b + ND, gbufs[b], gsems[b])

            @pl.loop(ND, NB - ND, step=ND)
            def _(j):
                for b in range(ND):
                    jj = j + b
                    wait_g(jj, gbufs[b], gsems[b])
                    wait_s(jj - ND, sbufs[b], ssems[b])
                    scale(jj, gbufs[b], sbufs[b])
                    issue_s(jj, sbufs[b], ssems[b])
                    issue_g(jj + ND, gbufs[b], gsems[b])

            for b in range(ND):
                jj = NB - ND + b
                wait_g(jj, gbufs[b], gsems[b])
                wait_s(jj - ND, sbufs[b], ssems[b])
                scale(jj, gbufs[b], sbufs[b])
                issue_s(jj, sbufs[b], ssems[b])
            for b in range(ND):
                wait_s(NB - ND + b, sbufs[b], ssems[b])

            plsc.subcore_barrier()
            pltpu.sync_copy(acc.at[pl.ds(base, ROW_SPAN)],
                            out_hbm.at[ti, cid, pl.ds(base, ROW_SPAN)])
            plsc.subcore_barrier()

    return _seg_kernel


_xside_kernel = _make_seg_kernel(2 * W, W, lambda ti: ti // 2, FH)
_hside_kernel = _make_seg_kernel(1, 1, lambda ti: ti, FH)


BN = 1000


def _prep_body(dp_ref, x_ref, dis_ref, xs_ref):
    dp = dp_ref[...]
    deg = dp[0, 0] + dp[0, 1]
    dis = jnp.where(deg > 0.0,
                    lax.rsqrt(jnp.maximum(deg, 1e-12)),
                    jnp.zeros_like(deg))
    dcol = dis[:, 0:1]
    dis_ref[...] = dcol.reshape(1, BN, 1)
    xb = x_ref[...][0]
    xs_ref[0] = dcol * xb[:, 0:FH]
    xs_ref[1] = dcol * xb[:, FH:FI]


def _prep_call(degparts, x):
    return pl.pallas_call(
        _prep_body,
        grid=(W, N // BN),
        in_specs=[
            pl.BlockSpec((1, NC, BN, 16), lambda t, i: (t, 0, i, 0)),
            pl.BlockSpec((1, BN, FI), lambda t, i: (t, i, 0)),
        ],
        out_specs=[
            pl.BlockSpec((1, BN, 1), lambda t, i: (t, i, 0)),
            pl.BlockSpec((2, BN, FH), lambda t, i: (t, i, 0)),
        ],
        out_shape=[
            jax.ShapeDtypeStruct((W, N, 1), jnp.float32),
            jax.ShapeDtypeStruct((2 * W, N, FH), jnp.float32),
        ],
    )(degparts, x)


def _gate_body(x_ref, u_ref, v_ref, h_ref, c_ref, dis_ref, disn_ref,
               wx0_ref, wx1_ref, wh0_ref, wh1_ref, b_ref, wc_ref, *rest):
    if len(rest) == 3:
        fin = ()
        hn_ref, cn_ref, hs_ref = rest
    else:
        lw_ref, lb_ref, hn_ref, cn_ref, hs_ref, o_ref = rest
        fin = (lw_ref, lb_ref, o_ref)
    d = dis_ref[...]
    ua = -d * (u_ref[0, 0] + u_ref[0, 1])
    ub = -d * (u_ref[1, 0] + u_ref[1, 1])
    v = -d * (v_ref[0, 0] + v_ref[0, 1])
    f32 = jnp.float32
    z = (jnp.dot(x_ref[...], wx0_ref[...], preferred_element_type=f32)
         + jnp.dot(ua, wx1_ref[...][:FH], preferred_element_type=f32)
         + jnp.dot(ub, wx1_ref[...][FH:], preferred_element_type=f32)
         + jnp.dot(h_ref[...], wh0_ref[...], preferred_element_type=f32)
         + jnp.dot(v, wh1_ref[...], preferred_element_type=f32)
         + b_ref[...])
    c_old = c_ref[...]
    ig = jax.nn.sigmoid(z[:, 0:FH] + wc_ref[0:1, :] * c_old)
    fg = jax.nn.sigmoid(z[:, FH:2 * FH] + wc_ref[1:2, :] * c_old)
    tg = jnp.tanh(z[:, 2 * FH:3 * FH])
    cn = fg * c_old + ig * tg
    og = jax.nn.sigmoid(z[:, 3 * FH:4 * FH] + wc_ref[2:3, :] * cn)
    hn = og * jnp.tanh(cn)
    hn_ref[...] = hn
    cn_ref[...] = cn
    hs_ref[...] = disn_ref[...] * hn
    if fin:
        o_ref = fin[2]
        o_ref[...] = (jnp.dot(hn, fin[0][...],
                              preferred_element_type=jnp.float32)
                      + fin[1][...])


def _gate_call(x_t, uparts, vparts, h, c, dis_t, dis_n, wx0, wx1,
               wh0, wh1, bias, wc3, lin=None):
    whole = lambda shp: pl.BlockSpec(shp, lambda i: tuple(0 for _ in shp))
    row = lambda f: pl.BlockSpec((BN, f), lambda i: (i, 0))
    in_specs = [
        row(FI),
        pl.BlockSpec((2, NC, BN, FH), lambda i: (0, 0, i, 0)),
        pl.BlockSpec((1, NC, BN, FH), lambda i: (0, 0, i, 0)),
        row(FH), row(FH), row(1), row(1),
        whole((FI, G4)), whole((FI, G4)),
        whole((FH, G4)), whole((FH, G4)),
        whole((1, G4)), whole((3, FH)),
    ]
    out_specs = [row(FH), row(FH), row(FH)]
    out_shape = [jax.ShapeDtypeStruct((N, FH), jnp.float32)] * 3
    args = [x_t, uparts, vparts, h, c, dis_t, dis_n, wx0, wx1, wh0, wh1,
            bias, wc3]
    if lin is not None:
        in_specs += [whole((FH, 1)), whole((1, 1))]
        out_specs.append(row(1))
        out_shape.append(jax.ShapeDtypeStruct((N, 1), jnp.float32))
        args += [lin[0], lin[1].reshape(1, 1)]
    return pl.pallas_call(
        _gate_body,
        grid=(N // BN,),
        in_specs=in_specs,
        out_specs=out_specs,
        out_shape=out_shape,
    )(*args)


def kernel(x, edge_index, edge_weight, Wx, Wh, bx, bh, wc, b, lin_W, lin_b):
    pad = E2 - E
    pidx = jnp.broadcast_to(jnp.arange(pad, dtype=jnp.int32) % N, (W, pad))
    src_r = jnp.concatenate([edge_index[:, 0, :], pidx],
                            axis=1).reshape(W, NW, NB, B)
    dst_r = jnp.concatenate([edge_index[:, 1, :], pidx],
                            axis=1).reshape(W, NW, NB, B)
    w_r = jnp.concatenate(
        [edge_weight, jnp.zeros((W, pad), jnp.float32)],
        axis=1).reshape(W, NW, EPW)

    wx0 = jnp.transpose(Wx[:, 0], (1, 0, 2)).reshape(FI, G4)
    wx1 = jnp.transpose(Wx[:, 1], (1, 0, 2)).reshape(FI, G4)
    wh0 = jnp.transpose(Wh[:, 0], (1, 0, 2)).reshape(FH, G4)
    wh1 = jnp.transpose(Wh[:, 1], (1, 0, 2)).reshape(FH, G4)
    bias = (bx + bh + b).reshape(1, G4)

    degparts = _deg_kernel(src_r, w_r)
    dis, xs = _prep_call(degparts, x)
    uparts = _xside_kernel(xs, src_r, dst_r, w_r)

    h = jnp.zeros((N, FH), jnp.float32)
    c = jnp.zeros((N, FH), jnp.float32)
    vzero = jnp.zeros((1, NC, N, FH), jnp.float32)
    hs = None
    for t in range(W):
        if t == 0:
            vparts = vzero
        else:
            vparts = _hside_kernel(hs[None], src_r[t:t + 1], dst_r[t:t + 1],
                                   w_r[t:t + 1])
        dis_n = dis[min(t + 1, W - 1)]
        lin = (lin_W, lin_b) if t == W - 1 else None
        res = _gate_call(x[t], uparts[2 * t:2 * t + 2], vparts, h, c,
                         dis[t], dis_n, wx0, wx1, wh0, wh1, bias, wc,
                         lin=lin)
        h, c, hs = res[0], res[1], res[2]
    out = res[3]
    return (out.reshape(N), h, c)
